# baseline (speedup 1.0000x reference)
import jax
import jax.numpy as jnp
from jax.experimental import pallas as pl
from jax.experimental.pallas import tpu as pltpu

_B = 256
_V = 50257
_U = 65536
_M = 3

_C = 512
_UB = 2048
_VP = ((_V + _C - 1) // _C) * _C
_CPM = _VP // _C
_NCHUNK = _M * _CPM
_NUB = _U // _UB
_BB = 32
_NB = _B // _BB


def _softmax_body(w_ref, x_ref, o_ref):
    x = x_ref[0]
    m = jnp.max(x, axis=-1, keepdims=True)
    e = jnp.exp(x - m)
    s = jnp.sum(e, axis=-1, keepdims=True)
    w = w_ref[0, :1, :1]
    p = e * (w / s)
    o_ref[...] = p.astype(jnp.bfloat16)[None]


def _scatter_body(map_ref, p_ref, o_ref):
    c = pl.program_id(1)
    ub = pl.program_id(0)

    @pl.when(c == 0)
    def _init():
        o_ref[...] = jnp.zeros_like(o_ref)

    mapv = map_ref[...]
    k = jax.lax.broadcasted_iota(jnp.int32, (_C, _UB), 1)
    sel = (mapv - ub * _UB) == k
    onehot = jnp.where(sel, jnp.float32(1.0),
                       jnp.float32(0.0)).astype(jnp.bfloat16)
    o_ref[...] += jnp.dot(p_ref[0], onehot,
                          preferred_element_type=jnp.float32)


def kernel(logits0, logits1, logits2, map0, map1, map2, weights):
    neg_inf = float("-inf")
    pad_w = _VP - _V
    lg = jnp.stack([
        jnp.pad(logits0, ((0, 0), (0, pad_w)), constant_values=neg_inf),
        jnp.pad(logits1, ((0, 0), (0, pad_w)), constant_values=neg_inf),
        jnp.pad(logits2, ((0, 0), (0, pad_w)), constant_values=neg_inf),
    ])
    w2 = jnp.broadcast_to(
        weights.astype(jnp.float32).reshape(_M, 1, 1), (_M, 8, 128))
    cmap = jnp.concatenate([
        jnp.pad(map0, (0, pad_w), constant_values=_U),
        jnp.pad(map1, (0, pad_w), constant_values=_U),
        jnp.pad(map2, (0, pad_w), constant_values=_U),
    ]).reshape(_M * _VP, 1)

    p = pl.pallas_call(
        _softmax_body,
        grid=(_NB, _M),
        in_specs=[
            pl.BlockSpec((1, 8, 128), lambda i, m: (m, 0, 0)),
            pl.BlockSpec((1, _BB, _VP), lambda i, m: (m, i, 0)),
        ],
        out_specs=pl.BlockSpec((1, _BB, _VP), lambda i, m: (m, i, 0)),
        out_shape=jax.ShapeDtypeStruct((_M, _B, _VP), jnp.bfloat16),
        compiler_params=pltpu.CompilerParams(
            dimension_semantics=("parallel", "parallel")),
        name="ens_softmax",
    )(w2, lg)

    union = pl.pallas_call(
        _scatter_body,
        grid=(_NUB, _NCHUNK),
        in_specs=[
            pl.BlockSpec((_C, 1), lambda ub, c: (c, 0)),
            pl.BlockSpec((1, _B, _C), lambda ub, c: (c // _CPM, 0, c % _CPM)),
        ],
        out_specs=pl.BlockSpec((_B, _UB), lambda ub, c: (0, ub)),
        out_shape=jax.ShapeDtypeStruct((_B, _U), jnp.float32),
        compiler_params=pltpu.CompilerParams(
            dimension_semantics=("parallel", "arbitrary")),
        name="ens_scatter",
    )(cmap, p)
    return union


# speedup vs baseline: 4.0230x; 4.0230x over previous
import jax
import jax.numpy as jnp
from jax.experimental import pallas as pl
from jax.experimental.pallas import tpu as pltpu

_B = 256
_V = 50257
_U = 65536
_M = 3

_CH = 1024
_VP = ((_V + _CH - 1) // _CH) * _CH
_CPM = _VP // _CH
_NCH = _M * _CPM
_HALF = _U // 2
_GRP = 8
_PIECE = 2048
_NFLUSH = _HALF // _PIECE


def _stats_body(x_ref, mx_ref, se_ref):
    c = pl.program_id(1)
    x = x_ref[0]

    @pl.when(c == 0)
    def _init():
        mx_ref[...] = jnp.full_like(mx_ref, -jnp.inf)
        se_ref[...] = jnp.zeros_like(se_ref)

    bm = jnp.max(x, axis=0, keepdims=True)
    m_old = mx_ref[0]
    m_new = jnp.maximum(m_old, bm)
    bs = jnp.sum(jnp.exp(x - m_new), axis=0, keepdims=True)
    se_ref[0] = se_ref[0] * jnp.exp(m_old - m_new) + bs
    mx_ref[0] = m_new


def _exp_body(w_ref, mx_ref, se_ref, x_ref, o_ref):
    x = x_ref[0]
    scale = w_ref[0] / se_ref[0]
    o_ref[0] = jnp.exp(x - mx_ref[0]) * scale


def _scatter_body(cmap_ref, pt_ref, o_ref, acc, idx_smem, sem):
    h = pl.program_id(0)
    c = pl.program_id(1)

    @pl.when(c < _NCH)
    def _accumulate():
        cp = pltpu.make_async_copy(cmap_ref.at[jnp.minimum(c, _NCH - 1)],
                                   idx_smem, sem)
        cp.start()

        @pl.when(c == 0)
        def _init():
            acc[...] = jnp.zeros_like(acc)

        cp.wait()
        base = h * _HALF

        def group(g, carry):
            rbase = g * _GRP
            for i in range(_GRP):
                r = rbase + i
                u = idx_smem[0, r] - base
                valid = jnp.logical_and(u >= 0, u < _HALF)
                a = jnp.where(valid, u, 0)
                scale = jnp.where(valid, jnp.float32(1.0), jnp.float32(0.0))
                acc[a] = acc[a] + pt_ref[0, r] * scale
            return carry

        jax.lax.fori_loop(0, _CH // _GRP, group, 0)

    @pl.when(c >= _NCH)
    def _flush():
        piece = c - _NCH
        o_ref[...] = acc[pl.ds(piece * _PIECE, _PIECE)]


def kernel(logits0, logits1, logits2, map0, map1, map2, weights):
    neg_inf = float("-inf")
    pad_w = _VP - _V
    lg = jnp.stack([
        jnp.pad(logits0, ((0, 0), (0, pad_w)), constant_values=neg_inf),
        jnp.pad(logits1, ((0, 0), (0, pad_w)), constant_values=neg_inf),
        jnp.pad(logits2, ((0, 0), (0, pad_w)), constant_values=neg_inf),
    ])
    lgt = jnp.transpose(lg, (0, 2, 1))
    w2 = jnp.broadcast_to(
        weights.astype(jnp.float32).reshape(_M, 1, 1), (_M, 1, _B))
    cmap = jnp.concatenate([
        jnp.pad(map0, (0, pad_w), constant_values=0),
        jnp.pad(map1, (0, pad_w), constant_values=0),
        jnp.pad(map2, (0, pad_w), constant_values=0),
    ]).reshape(_NCH, 1, _CH)

    mx, se = pl.pallas_call(
        _stats_body,
        grid=(_M, _CPM),
        in_specs=[pl.BlockSpec((1, _CH, _B), lambda m, c: (m, c, 0))],
        out_specs=[
            pl.BlockSpec((1, 1, _B), lambda m, c: (m, 0, 0)),
            pl.BlockSpec((1, 1, _B), lambda m, c: (m, 0, 0)),
        ],
        out_shape=[
            jax.ShapeDtypeStruct((_M, 1, _B), jnp.float32),
            jax.ShapeDtypeStruct((_M, 1, _B), jnp.float32),
        ],
        compiler_params=pltpu.CompilerParams(
            dimension_semantics=("parallel", "arbitrary")),
        name="ens_stats",
    )(lgt)

    pt = pl.pallas_call(
        _exp_body,
        grid=(_M, _CPM),
        in_specs=[
            pl.BlockSpec((1, 1, _B), lambda m, c: (m, 0, 0)),
            pl.BlockSpec((1, 1, _B), lambda m, c: (m, 0, 0)),
            pl.BlockSpec((1, 1, _B), lambda m, c: (m, 0, 0)),
            pl.BlockSpec((1, _CH, _B), lambda m, c: (m, c, 0)),
        ],
        out_specs=pl.BlockSpec((1, _CH, _B), lambda m, c: (m, c, 0)),
        out_shape=jax.ShapeDtypeStruct((_M, _VP, _B), jnp.float32),
        compiler_params=pltpu.CompilerParams(
            dimension_semantics=("parallel", "arbitrary")),
        name="ens_exp",
    )(w2, mx, se, lgt)

    pt4 = pt.reshape(_M, _VP, _B // 128, 128)

    uniont = pl.pallas_call(
        _scatter_body,
        grid=(2, _NCH + _NFLUSH),
        in_specs=[
            pl.BlockSpec(memory_space=pltpu.VMEM),
            pl.BlockSpec((1, _CH, _B // 128, 128),
                         lambda h, c: (jnp.minimum(c, _NCH - 1) // _CPM,
                                       jnp.minimum(c, _NCH - 1) % _CPM, 0, 0)),
        ],
        out_specs=pl.BlockSpec((_PIECE, _B // 128, 128),
                               lambda h, c: (h * _NFLUSH
                                             + jnp.maximum(c - _NCH, 0), 0, 0)),
        out_shape=jax.ShapeDtypeStruct((_U, _B // 128, 128), jnp.float32),
        scratch_shapes=[
            pltpu.VMEM((_HALF, _B // 128, 128), jnp.float32),
            pltpu.SMEM((1, _CH), jnp.int32),
            pltpu.SemaphoreType.DMA,
        ],
        compiler_params=pltpu.CompilerParams(
            dimension_semantics=("parallel", "arbitrary")),
        name="ens_scatter",
    )(cmap, pt4)

    return jnp.transpose(uniont.reshape(_U, _B), (1, 0))


# speedup vs baseline: 5.6019x; 1.3925x over previous
import jax
import jax.numpy as jnp
from jax.experimental import pallas as pl
from jax.experimental.pallas import tpu as pltpu

_B = 256
_V = 50257
_U = 65536
_M = 3

_CH = 1024
_VP = ((_V + _CH - 1) // _CH) * _CH
_CPM = _VP // _CH
_NCH = _M * _CPM
_HALF = _U // 2
_GRP = 8
_PIECE = 2048
_NFLUSH = _U // _PIECE


def _stats_body(x_ref, mx_ref, se_ref):
    c = pl.program_id(1)
    x = x_ref[0]

    @pl.when(c == 0)
    def _init():
        mx_ref[...] = jnp.full_like(mx_ref, -jnp.inf)
        se_ref[...] = jnp.zeros_like(se_ref)

    bm = jnp.max(x, axis=0, keepdims=True)
    m_old = mx_ref[0]
    m_new = jnp.maximum(m_old, bm)
    bs = jnp.sum(jnp.exp(x - m_new), axis=0, keepdims=True)
    se_ref[0] = se_ref[0] * jnp.exp(m_old - m_new) + bs
    mx_ref[0] = m_new


def _exp_body(w_ref, mx_ref, se_ref, x_ref, o_ref):
    x = x_ref[0]
    scale = w_ref[0] / se_ref[0]
    p = jnp.exp(x - mx_ref[0]) * scale
    o_ref[0, 0] = p[:, :128]
    o_ref[1, 0] = p[:, 128:]


def _scatter_body(cmap_ref, pt_ref, o_ref, acc, idx_smem, sem):
    c = pl.program_id(1)

    @pl.when(c < _NCH)
    def _accumulate():
        cp = pltpu.make_async_copy(cmap_ref.at[jnp.minimum(c, _NCH - 1)],
                                   idx_smem, sem)
        cp.start()

        @pl.when(c == 0)
        def _init():
            acc[...] = jnp.zeros_like(acc)

        cp.wait()

        def group(g, carry):
            rbase = g * _GRP
            for i in range(_GRP):
                r = rbase + i
                a = idx_smem[0, r]
                acc[a] = acc[a] + pt_ref[0, 0, r]
            return carry

        jax.lax.fori_loop(0, _CH // _GRP, group, 0)

    @pl.when(c >= _NCH)
    def _flush():
        piece = c - _NCH
        o_ref[0] = acc[pl.ds(piece * _PIECE, _PIECE), 0, :]


def kernel(logits0, logits1, logits2, map0, map1, map2, weights):
    neg_inf = float("-inf")
    pad_w = _VP - _V
    lg = jnp.stack([
        jnp.pad(logits0, ((0, 0), (0, pad_w)), constant_values=neg_inf),
        jnp.pad(logits1, ((0, 0), (0, pad_w)), constant_values=neg_inf),
        jnp.pad(logits2, ((0, 0), (0, pad_w)), constant_values=neg_inf),
    ])
    lgt = jnp.transpose(lg, (0, 2, 1))
    w2 = jnp.broadcast_to(
        weights.astype(jnp.float32).reshape(_M, 1, 1), (_M, 1, _B))
    cmap = jnp.concatenate([
        jnp.pad(map0, (0, pad_w), constant_values=0),
        jnp.pad(map1, (0, pad_w), constant_values=0),
        jnp.pad(map2, (0, pad_w), constant_values=0),
    ]).reshape(_NCH, 1, _CH)

    mx, se = pl.pallas_call(
        _stats_body,
        grid=(_M, _CPM),
        in_specs=[pl.BlockSpec((1, _CH, _B), lambda m, c: (m, c, 0))],
        out_specs=[
            pl.BlockSpec((1, 1, _B), lambda m, c: (m, 0, 0)),
            pl.BlockSpec((1, 1, _B), lambda m, c: (m, 0, 0)),
        ],
        out_shape=[
            jax.ShapeDtypeStruct((_M, 1, _B), jnp.float32),
            jax.ShapeDtypeStruct((_M, 1, _B), jnp.float32),
        ],
        compiler_params=pltpu.CompilerParams(
            dimension_semantics=("parallel", "arbitrary")),
        name="ens_stats",
    )(lgt)

    pt = pl.pallas_call(
        _exp_body,
        grid=(_M, _CPM),
        in_specs=[
            pl.BlockSpec((1, 1, _B), lambda m, c: (m, 0, 0)),
            pl.BlockSpec((1, 1, _B), lambda m, c: (m, 0, 0)),
            pl.BlockSpec((1, 1, _B), lambda m, c: (m, 0, 0)),
            pl.BlockSpec((1, _CH, _B), lambda m, c: (m, c, 0)),
        ],
        out_specs=pl.BlockSpec((2, 1, _CH, 128), lambda m, c: (0, m, c, 0)),
        out_shape=jax.ShapeDtypeStruct((2, _M, _VP, 128), jnp.float32),
        compiler_params=pltpu.CompilerParams(
            dimension_semantics=("parallel", "arbitrary")),
        name="ens_exp",
    )(w2, mx, se, lgt)

    uniont = pl.pallas_call(
        _scatter_body,
        grid=(2, _NCH + _NFLUSH),
        in_specs=[
            pl.BlockSpec(memory_space=pltpu.VMEM),
            pl.BlockSpec((1, 1, _CH, 128),
                         lambda h, c: (h, jnp.minimum(c, _NCH - 1) // _CPM,
                                       jnp.minimum(c, _NCH - 1) % _CPM, 0)),
        ],
        out_specs=pl.BlockSpec((1, _PIECE, 128),
                               lambda h, c: (h, jnp.maximum(c - _NCH, 0), 0)),
        out_shape=jax.ShapeDtypeStruct((2, _U, 128), jnp.float32),
        scratch_shapes=[
            pltpu.VMEM((_U, 1, 128), jnp.float32),
            pltpu.SMEM((1, _CH), jnp.int32),
            pltpu.SemaphoreType.DMA,
        ],
        compiler_params=pltpu.CompilerParams(
            dimension_semantics=("parallel", "arbitrary")),
        name="ens_scatter",
    )(cmap, pt)

    return jnp.transpose(uniont, (0, 2, 1)).reshape(_B, _U)
